# baseline (speedup 1.0000x reference)
import functools

import jax
import jax.numpy as jnp
import numpy as np
from jax import lax
from jax.experimental import pallas as pl
from jax.experimental.pallas import tpu as pltpu
from jax.experimental.pallas import tpu_sc as plsc

_F = 16
_LANES = 16
_V = 100000
_VBINS = _V + _LANES
_PAD_ID = _V + 1


def _sc_count_kernel(nc, rpt, lp, row_off, src_hbm, dst_hbm, zeros_hbm,
                     ws_hbm, wd_hbm, srows, drows, hist):
    nch = lp // _LANES
    wid = lax.axis_index("s") * nc + lax.axis_index("c")
    lbase = wid * rpt
    base = row_off + lbase

    pltpu.sync_copy(zeros_hbm, hist)
    pltpu.sync_copy(src_hbm.at[pl.ds(base, rpt)], srows)
    pltpu.sync_copy(dst_hbm.at[pl.ds(base, rpt)], drows)

    ones = jnp.full((_LANES,), 1, jnp.int32)
    hi_ones = jnp.full((_LANES,), 65536, jnp.int32)
    zeros16 = jnp.zeros((_LANES,), jnp.int32)

    def row_body(r, carry):
        s_chunks = [srows[r, pl.ds(c * _LANES, _LANES)] for c in range(nch)]
        d_chunks = [drows[r, pl.ds(c * _LANES, _LANES)] for c in range(nch)]
        for c in range(nch):
            plsc.addupdate_scatter(hist, [s_chunks[c]], ones)
        for c in range(nch):
            plsc.addupdate_scatter(hist, [d_chunks[c]], hi_ones)
        for c in range(nch):
            w = plsc.load_gather(hist, [s_chunks[c]])
            srows[r, pl.ds(c * _LANES, _LANES)] = jnp.where(
                s_chunks[c] == 0, 0, w)
        for c in range(nch):
            w = plsc.load_gather(hist, [d_chunks[c]])
            drows[r, pl.ds(c * _LANES, _LANES)] = jnp.where(
                d_chunks[c] == 0, 0, w)
        for c in range(nch):
            plsc.store_scatter(hist, [s_chunks[c]], zeros16)
        for c in range(nch):
            plsc.store_scatter(hist, [d_chunks[c]], zeros16)
        return carry

    lax.fori_loop(0, rpt, row_body, 0)

    pltpu.sync_copy(srows, ws_hbm.at[pl.ds(lbase, rpt)])
    pltpu.sync_copy(drows, wd_hbm.at[pl.ds(lbase, rpt)])


def _sc_counts(src_pad, dst_pad, zeros, row_off, nrows):
    _, lp = src_pad.shape
    info = plsc.get_sparse_core_info()
    nc, ns = info.num_cores, info.num_subcores
    nw = nc * ns
    rpt = nrows // nw
    mesh = plsc.VectorSubcoreMesh(core_axis_name="c", subcore_axis_name="s")
    out_t = jax.ShapeDtypeStruct((nrows, lp), jnp.int32)
    fn = pl.kernel(
        functools.partial(_sc_count_kernel, nc, rpt, lp, row_off),
        out_type=[out_t, out_t],
        mesh=mesh,
        scratch_types=[
            pltpu.VMEM((rpt, lp), jnp.int32),
            pltpu.VMEM((rpt, lp), jnp.int32),
            pltpu.VMEM((_VBINS,), jnp.int32),
        ],
        compiler_params=pltpu.CompilerParams(needs_layout_passes=False),
    )
    return fn(src_pad, dst_pad, zeros)


def _mlp_kernel2(l, ws_ref, wd_ref, rep_ref, w1t_ref, b1t_ref, d128_ref,
                 b2t_ref, os_in, od_in, os_ref, od_ref):
    del os_in, od_in
    _mlp_kernel(l, ws_ref, wd_ref, rep_ref, w1t_ref, b1t_ref, d128_ref,
                b2t_ref, os_ref, od_ref)


def _mlp_kernel(l, ws_ref, wd_ref, rep_ref, w1t_ref, b1t_ref, d128_ref,
                b2t_ref, os_ref, od_ref):
    rep = rep_ref[...]
    w1t = w1t_ref[...]
    b1t = b1t_ref[...]
    b2t = b2t_ref[...]
    d128 = d128_ref[...]

    def side(word, out_ref):
        word = word[:, :l]
        c1 = (word & 0xFFFF).astype(jnp.float32).astype(jnp.bfloat16)
        c2 = (word >> 16).astype(jnp.float32).astype(jnp.bfloat16)
        c1r = jnp.dot(c1, rep, preferred_element_type=jnp.float32)
        c2r = jnp.dot(c2, rep, preferred_element_type=jnp.float32)
        h = (jax.nn.relu(c1r * w1t + b1t)
             + jax.nn.relu(c2r * w1t + b1t))
        for t in range(l * _F // 128):
            lo, hi = t * 128, (t + 1) * 128
            out_ref[:, lo:hi] = (
                jnp.dot(h[:, lo:hi], d128,
                        preferred_element_type=jnp.float32)
                + b2t[:, lo:hi])

    side(ws_ref[...], os_ref)
    side(wd_ref[...], od_ref)


@jax.jit
def kernel(src_ids, dst_ids, W1, b1, W2, b2):
    b, l = src_ids.shape
    lp = -(-l // _LANES) * _LANES
    src_pad = jnp.pad(src_ids, ((0, 0), (0, lp - l)),
                      constant_values=_PAD_ID)
    dst_pad = jnp.pad(dst_ids, ((0, 0), (0, lp - l)),
                      constant_values=_PAD_ID)
    zeros = jnp.zeros((_VBINS,), jnp.int32)

    half = b // 2
    ws0, wd0 = _sc_counts(src_pad, dst_pad, zeros, 0, half)
    ws1, wd1 = _sc_counts(src_pad, dst_pad, zeros, half, half)

    lf = l * _F
    rep = jnp.repeat(jnp.eye(l, dtype=jnp.bfloat16), _F, axis=1)
    w1t = jnp.tile(W1.reshape(_F), l).reshape(1, lf)
    b1t = jnp.tile(b1, l).reshape(1, lf)
    b2t = jnp.tile(2.0 * b2, l).reshape(1, lf)
    d128 = jnp.kron(jnp.eye(128 // _F, dtype=jnp.float32), W2.T)

    r = 128
    hb = half // r
    cnt_spec = pl.BlockSpec((r, lp), lambda i: (i, 0))
    full = lambda shape: pl.BlockSpec(shape, lambda i: (0, 0))
    out_shape = jax.ShapeDtypeStruct((b, lf), jnp.float32)
    weight_specs = [full((l, lf)), full((1, lf)), full((1, lf)),
                    full((128, 128)), full((1, lf))]

    out_spec0 = pl.BlockSpec((r, lf), lambda i: (i, 0))
    os0, od0 = pl.pallas_call(
        functools.partial(_mlp_kernel, l),
        grid=(hb,),
        in_specs=[cnt_spec, cnt_spec] + weight_specs,
        out_specs=[out_spec0, out_spec0],
        out_shape=[out_shape, out_shape],
    )(ws0, wd0, rep, w1t, b1t, d128, b2t)

    out_spec1 = pl.BlockSpec((r, lf), lambda i: (i + hb, 0))
    dummy = pl.BlockSpec((8, 128), lambda i: (0, 0))
    os, od = pl.pallas_call(
        functools.partial(_mlp_kernel2, l),
        grid=(hb,),
        in_specs=[cnt_spec, cnt_spec] + weight_specs + [dummy, dummy],
        out_specs=[out_spec1, out_spec1],
        out_shape=[out_shape, out_shape],
        input_output_aliases={7: 0, 8: 1},
    )(ws1, wd1, rep, w1t, b1t, d128, b2t, os0, od0)
    return (os.reshape(b, l, _F), od.reshape(b, l, _F))


# speedup vs baseline: 1.1548x; 1.1548x over previous
import functools

import jax
import jax.numpy as jnp
import numpy as np
from jax import lax
from jax.experimental import pallas as pl
from jax.experimental.pallas import tpu as pltpu
from jax.experimental.pallas import tpu_sc as plsc

_F = 16
_LANES = 16
_V = 100000
_VBINS = _V + _LANES
_PAD_ID = _V + 1


def _sc_count_kernel(nc, rpt, l, lp, src_hbm, dst_hbm, zeros_hbm, ws_hbm,
                     wd_hbm, sflat, dflat, wsrows, wdrows, hist):
    nch = lp // _LANES
    nfull = l // _LANES
    wid = lax.axis_index("s") * nc + lax.axis_index("c")
    base = wid * rpt

    pltpu.sync_copy(zeros_hbm, hist)
    pltpu.sync_copy(src_hbm.at[pl.ds(base * l, rpt * l)],
                    sflat.at[pl.ds(0, rpt * l)])
    pltpu.sync_copy(dst_hbm.at[pl.ds(base * l, rpt * l)],
                    dflat.at[pl.ds(0, rpt * l)])

    ones = jnp.full((_LANES,), 1, jnp.int32)
    hi_ones = jnp.full((_LANES,), 65536, jnp.int32)
    zeros16 = jnp.zeros((_LANES,), jnp.int32)
    tail = lax.iota(jnp.int32, _LANES) < (l - nfull * _LANES)
    masks = [None] * nfull + [tail] * (nch - nfull)

    def row_body(r, carry):
        s_chunks = [sflat[pl.ds(r * l + c * _LANES, _LANES)]
                    for c in range(nch)]
        d_chunks = [dflat[pl.ds(r * l + c * _LANES, _LANES)]
                    for c in range(nch)]
        for c in range(nch):
            plsc.addupdate_scatter(hist, [s_chunks[c]], ones, mask=masks[c])
        for c in range(nch):
            plsc.addupdate_scatter(hist, [d_chunks[c]], hi_ones,
                                   mask=masks[c])
        for c in range(nch):
            w = plsc.load_gather(hist, [s_chunks[c]], mask=masks[c])
            wsrows[r, pl.ds(c * _LANES, _LANES)] = jnp.where(
                s_chunks[c] == 0, 0, w)
        for c in range(nch):
            w = plsc.load_gather(hist, [d_chunks[c]], mask=masks[c])
            wdrows[r, pl.ds(c * _LANES, _LANES)] = jnp.where(
                d_chunks[c] == 0, 0, w)
        for c in range(nch):
            plsc.store_scatter(hist, [s_chunks[c]], zeros16, mask=masks[c])
        for c in range(nch):
            plsc.store_scatter(hist, [d_chunks[c]], zeros16, mask=masks[c])
        return carry

    lax.fori_loop(0, rpt, row_body, 0)

    pltpu.sync_copy(wsrows, ws_hbm.at[pl.ds(base, rpt)])
    pltpu.sync_copy(wdrows, wd_hbm.at[pl.ds(base, rpt)])


def _sc_counts(src_flat, dst_flat, zeros, b, l):
    lp = -(-l // _LANES) * _LANES
    info = plsc.get_sparse_core_info()
    nc, ns = info.num_cores, info.num_subcores
    nw = nc * ns
    rpt = b // nw
    mesh = plsc.VectorSubcoreMesh(core_axis_name="c", subcore_axis_name="s")
    out_t = jax.ShapeDtypeStruct((b, lp), jnp.int32)
    fn = pl.kernel(
        functools.partial(_sc_count_kernel, nc, rpt, l, lp),
        out_type=[out_t, out_t],
        mesh=mesh,
        scratch_types=[
            pltpu.VMEM((rpt * l + _LANES,), jnp.int32),
            pltpu.VMEM((rpt * l + _LANES,), jnp.int32),
            pltpu.VMEM((rpt, lp), jnp.int32),
            pltpu.VMEM((rpt, lp), jnp.int32),
            pltpu.VMEM((_VBINS,), jnp.int32),
        ],
        compiler_params=pltpu.CompilerParams(needs_layout_passes=False),
    )
    return fn(src_flat, dst_flat, zeros)


def _mlp_kernel(l, ws_ref, wd_ref, rep_ref, w1t_ref, b1t_ref, d128_ref,
                b2t_ref, os_ref, od_ref):
    rep = rep_ref[...]
    w1t = w1t_ref[...]
    b1t = b1t_ref[...]
    b2t = b2t_ref[...]
    d128 = d128_ref[...]

    def side(word, out_ref):
        word = word[:, :l]
        c1 = (word & 0xFFFF).astype(jnp.float32).astype(jnp.bfloat16)
        c2 = (word >> 16).astype(jnp.float32).astype(jnp.bfloat16)
        c1r = jnp.dot(c1, rep, preferred_element_type=jnp.float32)
        c2r = jnp.dot(c2, rep, preferred_element_type=jnp.float32)
        h = (jax.nn.relu(c1r * w1t + b1t)
             + jax.nn.relu(c2r * w1t + b1t))
        for t in range(l * _F // 128):
            lo, hi = t * 128, (t + 1) * 128
            out_ref[:, lo:hi] = (
                jnp.dot(h[:, lo:hi], d128,
                        preferred_element_type=jnp.float32)
                + b2t[:, lo:hi])

    side(ws_ref[...], os_ref)
    side(wd_ref[...], od_ref)


@jax.jit
def kernel(src_ids, dst_ids, W1, b1, W2, b2):
    b, l = src_ids.shape
    lp = -(-l // _LANES) * _LANES
    zeros = jnp.zeros((_VBINS,), jnp.int32)

    ws, wd = _sc_counts(src_ids.reshape(b * l), dst_ids.reshape(b * l),
                        zeros, b, l)

    lf = l * _F
    rep = jnp.repeat(jnp.eye(l, dtype=jnp.bfloat16), _F, axis=1)
    w1t = jnp.tile(W1.reshape(_F), l).reshape(1, lf)
    b1t = jnp.tile(b1, l).reshape(1, lf)
    b2t = jnp.tile(2.0 * b2, l).reshape(1, lf)
    d128 = jnp.kron(jnp.eye(128 // _F, dtype=jnp.float32), W2.T)

    r = 256
    cnt_spec = pl.BlockSpec((r, lp), lambda i: (i, 0))
    full = lambda shape: pl.BlockSpec(shape, lambda i: (0, 0))
    out_spec = pl.BlockSpec((r, lf), lambda i: (i, 0))
    out_shape = jax.ShapeDtypeStruct((b, lf), jnp.float32)

    os, od = pl.pallas_call(
        functools.partial(_mlp_kernel, l),
        grid=(b // r,),
        in_specs=[cnt_spec, cnt_spec,
                  full((l, lf)), full((1, lf)), full((1, lf)),
                  full((128, 128)), full((1, lf))],
        out_specs=[out_spec, out_spec],
        out_shape=[out_shape, out_shape],
    )(ws, wd, rep, w1t, b1t, d128, b2t)
    return (os.reshape(b, l, _F), od.reshape(b, l, _F))


# speedup vs baseline: 1.1720x; 1.0149x over previous
import functools

import jax
import jax.numpy as jnp
from jax import lax
from jax.experimental import pallas as pl
from jax.experimental.pallas import tpu as pltpu
from jax.experimental.pallas import tpu_sc as plsc

_F = 16
_LANES = 16
_V = 100000
_VBINS = _V + _LANES
_PAD_ID = _V + 1


def _sc_count_kernel(nc, rpt, lp, src_hbm, dst_hbm, zeros_hbm, ws_hbm,
                     wd_hbm, srows, drows, hist):
    nch = lp // _LANES
    wid = lax.axis_index("s") * nc + lax.axis_index("c")
    base = wid * rpt

    pltpu.sync_copy(zeros_hbm, hist)
    pltpu.sync_copy(src_hbm.at[pl.ds(base, rpt)], srows)
    pltpu.sync_copy(dst_hbm.at[pl.ds(base, rpt)], drows)

    ones = jnp.full((_LANES,), 1, jnp.int32)
    hi_ones = jnp.full((_LANES,), 65536, jnp.int32)
    zeros16 = jnp.zeros((_LANES,), jnp.int32)

    def row_body(r, carry):
        s_chunks = [srows[r, pl.ds(c * _LANES, _LANES)] for c in range(nch)]
        d_chunks = [drows[r, pl.ds(c * _LANES, _LANES)] for c in range(nch)]
        for c in range(nch):
            plsc.addupdate_scatter(hist, [s_chunks[c]], ones)
        for c in range(nch):
            plsc.addupdate_scatter(hist, [d_chunks[c]], hi_ones)
        for c in range(nch):
            w = plsc.load_gather(hist, [s_chunks[c]])
            srows[r, pl.ds(c * _LANES, _LANES)] = jnp.where(
                s_chunks[c] == 0, 0, w)
        for c in range(nch):
            w = plsc.load_gather(hist, [d_chunks[c]])
            drows[r, pl.ds(c * _LANES, _LANES)] = jnp.where(
                d_chunks[c] == 0, 0, w)
        for c in range(nch):
            plsc.store_scatter(hist, [s_chunks[c]], zeros16)
        for c in range(nch):
            plsc.store_scatter(hist, [d_chunks[c]], zeros16)
        return carry

    lax.fori_loop(0, rpt, row_body, 0)

    pltpu.sync_copy(srows, ws_hbm.at[pl.ds(base, rpt)])
    pltpu.sync_copy(drows, wd_hbm.at[pl.ds(base, rpt)])


def _sc_counts(src_pad, dst_pad, zeros):
    b, lp = src_pad.shape
    info = plsc.get_sparse_core_info()
    nc, ns = info.num_cores, info.num_subcores
    nw = nc * ns
    rpt = b // nw
    mesh = plsc.VectorSubcoreMesh(core_axis_name="c", subcore_axis_name="s")
    out_t = jax.ShapeDtypeStruct((b, lp), jnp.int32)
    fn = pl.kernel(
        functools.partial(_sc_count_kernel, nc, rpt, lp),
        out_type=[out_t, out_t],
        mesh=mesh,
        scratch_types=[
            pltpu.VMEM((rpt, lp), jnp.int32),
            pltpu.VMEM((rpt, lp), jnp.int32),
            pltpu.VMEM((_VBINS,), jnp.int32),
        ],
        compiler_params=pltpu.CompilerParams(needs_layout_passes=False),
    )
    return fn(src_pad, dst_pad, zeros)


def _mlp_kernel(l, ws_ref, wd_ref, rep_ref, w1t_ref, b1t_ref, d128_ref,
                b2t_ref, os_ref, od_ref):
    rep = rep_ref[...]
    w1t = w1t_ref[...]
    b1t = b1t_ref[...]
    b2t = b2t_ref[...]
    d128 = d128_ref[...]

    def side(word, out_ref):
        word = word[:, :l]
        c1 = (word & 0xFFFF).astype(jnp.float32).astype(jnp.bfloat16)
        c2 = (word >> 16).astype(jnp.float32).astype(jnp.bfloat16)
        c1r = jnp.dot(c1, rep, preferred_element_type=jnp.float32)
        c2r = jnp.dot(c2, rep, preferred_element_type=jnp.float32)
        h = (jax.nn.relu(c1r * w1t + b1t)
             + jax.nn.relu(c2r * w1t + b1t))
        for t in range(l * _F // 128):
            lo, hi = t * 128, (t + 1) * 128
            out_ref[:, lo:hi] = (
                jnp.dot(h[:, lo:hi], d128,
                        preferred_element_type=jnp.float32)
                + b2t[:, lo:hi])

    side(ws_ref[...], os_ref)
    side(wd_ref[...], od_ref)


@jax.jit
def kernel(src_ids, dst_ids, W1, b1, W2, b2):
    b, l = src_ids.shape
    lp = -(-l // _LANES) * _LANES
    src_pad = jnp.pad(src_ids, ((0, 0), (0, lp - l)),
                      constant_values=_PAD_ID)
    dst_pad = jnp.pad(dst_ids, ((0, 0), (0, lp - l)),
                      constant_values=_PAD_ID)
    zeros = jnp.zeros((_VBINS,), jnp.int32)

    ws, wd = _sc_counts(src_pad, dst_pad, zeros)

    lf = l * _F
    rep = jnp.repeat(jnp.eye(l, dtype=jnp.bfloat16), _F, axis=1)
    w1t = jnp.tile(W1.reshape(_F), l).reshape(1, lf)
    b1t = jnp.tile(b1, l).reshape(1, lf)
    b2t = jnp.tile(2.0 * b2, l).reshape(1, lf)
    d128 = jnp.kron(jnp.eye(128 // _F, dtype=jnp.float32), W2.T)

    r = 256
    cnt_spec = pl.BlockSpec((r, lp), lambda i: (i, 0))
    full = lambda shape: pl.BlockSpec(shape, lambda i: (0, 0))
    out_spec = pl.BlockSpec((r, lf), lambda i: (i, 0))
    out_shape = jax.ShapeDtypeStruct((b, lf), jnp.float32)

    os, od = pl.pallas_call(
        functools.partial(_mlp_kernel, l),
        grid=(b // r,),
        in_specs=[cnt_spec, cnt_spec,
                  full((l, lf)), full((1, lf)), full((1, lf)),
                  full((128, 128)), full((1, lf))],
        out_specs=[out_spec, out_spec],
        out_shape=[out_shape, out_shape],
    )(ws, wd, rep, w1t, b1t, d128, b2t)
    return (os.reshape(b, l, _F), od.reshape(b, l, _F))
